# baseline (speedup 1.0000x reference)
import functools

import jax
import jax.numpy as jnp
from jax import lax
from jax.experimental import pallas as pl
from jax.experimental.pallas import tpu as pltpu
from jax.experimental.pallas import tpu_sc as plsc

N = 10000
E = 320000
D = 128
H = 16
C = 7
K_HOPS = 10
ALPHA = 0.1

NC = 2
NS = 16
NW = NC * NS
EPW = E // NW
BLK = 128
NBLK = 80
EPWP = NBLK * BLK
NP = 10240
RPT = NP // NS
NPK = NP // 8
PAD_SRC = 0
PAD_DST = (N % NPK) * 8 + N // NPK


def _sc_mesh():
    return plsc.VectorSubcoreMesh(
        core_axis_name="c", subcore_axis_name="s", num_cores=NC, num_subcores=NS
    )


def _zero_acc_slice(zbuf, acc, s):
    def body(i, _):
        zbuf[i, :] = jnp.zeros((H,), jnp.float32)
        return 0
    lax.fori_loop(0, RPT, body, 0)
    pltpu.sync_copy(zbuf, acc.at[pl.ds(s * RPT, RPT)])


def _deg_body(dst_hbm, out_hbm, dst_v, ones_v, zbuf, acc, ssem):
    c = lax.axis_index("c")
    s = lax.axis_index("s")
    wid = c * NS + s
    _zero_acc_slice(zbuf, acc, s)

    def fill_ones(j, _):
        ones_v[j, :] = jnp.ones((H,), jnp.float32)
        return 0
    lax.fori_loop(0, BLK, fill_ones, 0)
    pltpu.sync_copy(dst_hbm.at[wid], dst_v)
    plsc.subcore_barrier()

    def blk(j, _):
        pltpu.async_copy(ones_v, acc.at[dst_v.at[j]], ssem, add=True)
        return 0
    lax.fori_loop(0, NBLK, blk, 0)

    def drain(j, _):
        pltpu.make_async_copy(ones_v, acc.at[dst_v.at[j]], ssem).wait()
        return 0
    lax.fori_loop(0, NBLK, drain, 0)
    plsc.subcore_barrier()
    pltpu.sync_copy(acc.at[pl.ds(s * RPT, RPT)], out_hbm.at[c, pl.ds(s * RPT, RPT)])


_deg_kernel = pl.kernel(
    _deg_body,
    out_type=jax.ShapeDtypeStruct((NC, NP, H), jnp.float32),
    mesh=_sc_mesh(),
    compiler_params=pltpu.CompilerParams(use_tc_tiling_on_sc=False),
    scratch_types=[
        pltpu.VMEM((NBLK, BLK), jnp.int32),
        pltpu.VMEM((BLK, H), jnp.float32),
        pltpu.VMEM((RPT, H), jnp.float32),
        pltpu.VMEM_SHARED((NP, H), jnp.float32),
        pltpu.SemaphoreType.DMA,
    ],
)


G = 5
NG = NBLK // G


def _prop_body(y_hbm, src_hbm, dst_hbm, out_hbm, src_v, dst_v, rows_v, zbuf, acc,
               gsem_a, gsem_b, ssem_a, ssem_b):
    c = lax.axis_index("c")
    s = lax.axis_index("s")
    wid = c * NS + s
    _zero_acc_slice(zbuf, acc, s)
    pltpu.sync_copy(src_hbm.at[wid], src_v)
    pltpu.sync_copy(dst_hbm.at[wid], dst_v)
    plsc.subcore_barrier()

    def gather_start(j, p, b, sem):
        pltpu.async_copy(y_hbm.at[src_v.at[j]], rows_v.at[p, b], sem)

    def gather_wait(j, p, b, sem):
        pltpu.make_async_copy(y_hbm.at[src_v.at[j]], rows_v.at[p, b], sem).wait()

    def scat_start(j, p, b, sem):
        pltpu.async_copy(rows_v.at[p, b], acc.at[dst_v.at[j]], sem, add=True)

    def scat_wait(j, p, b, sem):
        pltpu.make_async_copy(rows_v.at[p, b], acc.at[dst_v.at[j]], sem).wait()

    for b in range(G):
        gather_start(b, 0, b, gsem_a)

    def pair(g, _):
        ja = 2 * g * G
        jb = ja + G
        jn = jb + G
        for b in range(G):
            gather_wait(ja + b, 0, b, gsem_a)
        for b in range(G):
            scat_start(ja + b, 0, b, ssem_a)

        @pl.when(g > 0)
        def _():
            for b in range(G):
                scat_wait(ja - G + b, 1, b, ssem_b)
        for b in range(G):
            gather_start(jb + b, 1, b, gsem_b)
        for b in range(G):
            gather_wait(jb + b, 1, b, gsem_b)
        for b in range(G):
            scat_start(jb + b, 1, b, ssem_b)
        for b in range(G):
            scat_wait(ja + b, 0, b, ssem_a)

        @pl.when(jn < NBLK)
        def _():
            for b in range(G):
                gather_start(jn + b, 0, b, gsem_a)
        return 0

    lax.fori_loop(0, NG // 2, pair, 0)
    for b in range(G):
        scat_wait(NBLK - G + b, 1, b, ssem_b)
    plsc.subcore_barrier()
    pltpu.sync_copy(acc.at[pl.ds(s * RPT, RPT)], out_hbm.at[c, pl.ds(s * RPT, RPT)])


_prop_kernel = pl.kernel(
    _prop_body,
    out_type=jax.ShapeDtypeStruct((NC, NP, H), jnp.float32),
    mesh=_sc_mesh(),
    compiler_params=pltpu.CompilerParams(use_tc_tiling_on_sc=False),
    scratch_types=[
        pltpu.VMEM((NBLK, BLK), jnp.int32),
        pltpu.VMEM((NBLK, BLK), jnp.int32),
        pltpu.VMEM((2, G, BLK, H), jnp.float32),
        pltpu.VMEM((RPT, H), jnp.float32),
        pltpu.VMEM_SHARED((NP, H), jnp.float32),
        pltpu.SemaphoreType.DMA,
        pltpu.SemaphoreType.DMA,
        pltpu.SemaphoreType.DMA,
        pltpu.SemaphoreType.DMA,
    ],
)


def _blkdiag(w):
    return (jnp.eye(8, dtype=w.dtype)[:, None, :, None]
            * w[None, :, None, :]).reshape(H * 8, H * 8)


def _tc1_body(x_ref, w1_ref, d_ref, ys_ref, dis_ref):
    xw = jnp.dot(x_ref[...], w1_ref[...], preferred_element_type=jnp.float32)
    xw_full = jnp.concatenate([xw, jnp.zeros((NP - N, H), jnp.float32)])
    xw_p = jnp.concatenate(
        [xw_full[NPK * r:NPK * (r + 1)] for r in range(8)], axis=1)
    deg = 1.0 + d_ref[0] + d_ref[1]
    dis = lax.rsqrt(deg)
    ys_ref[...] = xw_p * dis
    dis_ref[...] = dis


def _tc2_body(p_ref, ys_ref, dis_ref, b1_ref, wl_ref, bl_ref, w2_ref, zs_ref):
    dis = dis_ref[...]
    h = dis * (p_ref[0] + p_ref[1] + ys_ref[...]) + b1_ref[...]
    h = jnp.maximum(h, 0.0)
    wl = wl_ref[...]
    bl = bl_ref[...]
    for _ in range(K_HOPS):
        t = jnp.dot(h, wl, preferred_element_type=jnp.float32) + bl
        h = ALPHA * jnp.maximum(t, 0.0) + (1.0 - ALPHA) * h
    z = jnp.dot(h, w2_ref[...], preferred_element_type=jnp.float32)
    zs_ref[...] = z * dis


def _tc3_body(q_ref, zs_ref, dis_ref, b2_ref, out_ref):
    o_p = dis_ref[...] * (q_ref[0] + q_ref[1] + zs_ref[...])
    o_full = jnp.concatenate(
        [o_p[:, H * r:H * (r + 1)] for r in range(8)], axis=0)
    o = o_full[:N, :C] + b2_ref[...]
    m = jnp.max(o, axis=1, keepdims=True)
    lse = jnp.log(jnp.sum(jnp.exp(o - m), axis=1, keepdims=True))
    out_ref[...] = o - m - lse


_tc1 = pl.pallas_call(
    _tc1_body,
    out_shape=[
        jax.ShapeDtypeStruct((NPK, 128), jnp.float32),
        jax.ShapeDtypeStruct((NPK, 128), jnp.float32),
    ],
)

_tc2 = pl.pallas_call(
    _tc2_body,
    out_shape=jax.ShapeDtypeStruct((NPK, 128), jnp.float32),
)

_tc3 = pl.pallas_call(
    _tc3_body,
    out_shape=jax.ShapeDtypeStruct((N, C), jnp.float32),
)


@jax.jit
def kernel(x, edge_index, W1, b1, Wl, bl, W2, b2):
    def _prep(row, padval):
        t = (row % NPK) * 8 + row // NPK
        t = jnp.pad(t.reshape(NW, EPW), ((0, 0), (0, EPWP - EPW)),
                    constant_values=padval)
        return t.reshape(NW, NBLK, BLK)

    src_r = _prep(edge_index[0], PAD_SRC)
    dst_r = _prep(edge_index[1], PAD_DST)
    wl_blk = _blkdiag(Wl)
    w2_blk = _blkdiag(jnp.pad(W2, ((0, 0), (0, H - C))))
    b1_p = jnp.tile(b1, 8).reshape(1, 128)
    bl_p = jnp.tile(bl, 8).reshape(1, 128)

    deg16 = _deg_kernel(dst_r)
    deg_p = deg16.reshape(NC, NPK, 128)
    ys_p, dis_p = _tc1(x, W1, deg_p)
    ys = ys_p.reshape(NP, H)
    p = _prop_kernel(ys, src_r, dst_r)
    zs_p = _tc2(p.reshape(NC, NPK, 128), ys_p, dis_p, b1_p, wl_blk, bl_p, w2_blk)
    q = _prop_kernel(zs_p.reshape(NP, H), src_r, dst_r)
    return _tc3(q.reshape(NC, NPK, 128), zs_p, dis_p, b2.reshape(1, C))


# speedup vs baseline: 1.1654x; 1.1654x over previous
import functools

import jax
import jax.numpy as jnp
import numpy as np
from jax import lax
from jax.experimental import pallas as pl
from jax.experimental.pallas import tpu as pltpu
from jax.experimental.pallas import tpu_sc as plsc

N = 10000
E = 320000
D = 128
H = 16
C = 7
K_HOPS = 10
ALPHA = 0.1

NC = 2
NS = 16
NW = NC * NS
EPW = E // NW
BLK = 128
NBLK = 80
EPWP = NBLK * BLK
NP = 10240
RPT = NP // NS
NPK = NP // 8
_PAD_IDS = np.arange(N, NP)
PAD_SRC = np.zeros(EPWP - EPW, np.int32)
PAD_DST = ((_PAD_IDS % NPK) * 8 + _PAD_IDS // NPK).astype(np.int32)


def _sc_mesh():
    return plsc.VectorSubcoreMesh(
        core_axis_name="c", subcore_axis_name="s", num_cores=NC, num_subcores=NS
    )


def _zero_acc_slice(zbuf, acc, s):
    def body(i, _):
        zbuf[i, :] = jnp.zeros((H,), jnp.float32)
        return 0
    lax.fori_loop(0, RPT, body, 0)
    pltpu.sync_copy(zbuf, acc.at[pl.ds(s * RPT, RPT)])


def _deg_body(dst_hbm, out_hbm, dst_v, ones_v, zbuf, acc, ssem):
    c = lax.axis_index("c")
    s = lax.axis_index("s")
    wid = c * NS + s
    _zero_acc_slice(zbuf, acc, s)

    def fill_ones(j, _):
        ones_v[j, :] = jnp.ones((H,), jnp.float32)
        return 0
    lax.fori_loop(0, BLK, fill_ones, 0)
    pltpu.sync_copy(dst_hbm.at[wid], dst_v)
    plsc.subcore_barrier()

    def blk(j, _):
        pltpu.async_copy(ones_v, acc.at[dst_v.at[j]], ssem, add=True)
        return 0
    lax.fori_loop(0, NBLK, blk, 0)

    def drain(j, _):
        pltpu.make_async_copy(ones_v, acc.at[dst_v.at[j]], ssem).wait()
        return 0
    lax.fori_loop(0, NBLK, drain, 0)
    plsc.subcore_barrier()
    pltpu.sync_copy(acc.at[pl.ds(s * RPT, RPT)], out_hbm.at[c, pl.ds(s * RPT, RPT)])


_deg_kernel = pl.kernel(
    _deg_body,
    out_type=jax.ShapeDtypeStruct((NC, NP, H), jnp.float32),
    mesh=_sc_mesh(),
    compiler_params=pltpu.CompilerParams(use_tc_tiling_on_sc=False),
    scratch_types=[
        pltpu.VMEM((NBLK, BLK), jnp.int32),
        pltpu.VMEM((BLK, H), jnp.float32),
        pltpu.VMEM((RPT, H), jnp.float32),
        pltpu.VMEM_SHARED((NP, H), jnp.float32),
        pltpu.SemaphoreType.DMA,
    ],
)


G = 5
NG = NBLK // G


def _prop_body(y_hbm, src_hbm, dst_hbm, out_hbm, src_v, dst_v, rows_v, zbuf, acc,
               gsem_a, gsem_b, ssem_a, ssem_b):
    c = lax.axis_index("c")
    s = lax.axis_index("s")
    wid = c * NS + s
    _zero_acc_slice(zbuf, acc, s)
    pltpu.sync_copy(src_hbm.at[wid], src_v)
    pltpu.sync_copy(dst_hbm.at[wid], dst_v)
    plsc.subcore_barrier()

    def gather_start(j, p, b, sem):
        pltpu.async_copy(y_hbm.at[src_v.at[j]], rows_v.at[p, b], sem)

    def gather_wait(j, p, b, sem):
        pltpu.make_async_copy(y_hbm.at[src_v.at[j]], rows_v.at[p, b], sem).wait()

    def scat_start(j, p, b, sem):
        pltpu.async_copy(rows_v.at[p, b], acc.at[dst_v.at[j]], sem, add=True)

    def scat_wait(j, p, b, sem):
        pltpu.make_async_copy(rows_v.at[p, b], acc.at[dst_v.at[j]], sem).wait()

    for b in range(G):
        gather_start(b, 0, b, gsem_a)

    def pair(g, _):
        ja = 2 * g * G
        jb = ja + G
        jn = jb + G
        for b in range(G):
            gather_wait(ja + b, 0, b, gsem_a)
        for b in range(G):
            scat_start(ja + b, 0, b, ssem_a)

        @pl.when(g > 0)
        def _():
            for b in range(G):
                scat_wait(ja - G + b, 1, b, ssem_b)
        for b in range(G):
            gather_start(jb + b, 1, b, gsem_b)
        for b in range(G):
            gather_wait(jb + b, 1, b, gsem_b)
        for b in range(G):
            scat_start(jb + b, 1, b, ssem_b)
        for b in range(G):
            scat_wait(ja + b, 0, b, ssem_a)

        @pl.when(jn < NBLK)
        def _():
            for b in range(G):
                gather_start(jn + b, 0, b, gsem_a)
        return 0

    lax.fori_loop(0, NG // 2, pair, 0)
    for b in range(G):
        scat_wait(NBLK - G + b, 1, b, ssem_b)
    plsc.subcore_barrier()
    pltpu.sync_copy(acc.at[pl.ds(s * RPT, RPT)], out_hbm.at[c, pl.ds(s * RPT, RPT)])


_prop_kernel = pl.kernel(
    _prop_body,
    out_type=jax.ShapeDtypeStruct((NC, NP, H), jnp.float32),
    mesh=_sc_mesh(),
    compiler_params=pltpu.CompilerParams(use_tc_tiling_on_sc=False),
    scratch_types=[
        pltpu.VMEM((NBLK, BLK), jnp.int32),
        pltpu.VMEM((NBLK, BLK), jnp.int32),
        pltpu.VMEM((2, G, BLK, H), jnp.float32),
        pltpu.VMEM((RPT, H), jnp.float32),
        pltpu.VMEM_SHARED((NP, H), jnp.float32),
        pltpu.SemaphoreType.DMA,
        pltpu.SemaphoreType.DMA,
        pltpu.SemaphoreType.DMA,
        pltpu.SemaphoreType.DMA,
    ],
)


def _blkdiag(w):
    return (jnp.eye(8, dtype=w.dtype)[:, None, :, None]
            * w[None, :, None, :]).reshape(H * 8, H * 8)


def _tc1_body(x_ref, w1_ref, d_ref, ys_ref, dis_ref):
    xw = jnp.dot(x_ref[...], w1_ref[...], preferred_element_type=jnp.float32)
    xw_full = jnp.concatenate([xw, jnp.zeros((NP - N, H), jnp.float32)])
    xw_p = jnp.concatenate(
        [xw_full[NPK * r:NPK * (r + 1)] for r in range(8)], axis=1)
    deg = 1.0 + d_ref[0] + d_ref[1]
    dis = lax.rsqrt(deg)
    ys_ref[...] = xw_p * dis
    dis_ref[...] = dis


def _tc2_body(p_ref, ys_ref, dis_ref, b1_ref, wl_ref, bl_ref, w2_ref, zs_ref):
    dis = dis_ref[...]
    h = dis * (p_ref[0] + p_ref[1] + ys_ref[...]) + b1_ref[...]
    h = jnp.maximum(h, 0.0)
    wl = wl_ref[...]
    bl = bl_ref[...]
    for _ in range(K_HOPS):
        t = jnp.dot(h, wl, preferred_element_type=jnp.float32) + bl
        h = ALPHA * jnp.maximum(t, 0.0) + (1.0 - ALPHA) * h
    z = jnp.dot(h, w2_ref[...], preferred_element_type=jnp.float32)
    zs_ref[...] = z * dis


def _tc3_body(q_ref, zs_ref, dis_ref, b2_ref, out_ref):
    o_p = dis_ref[...] * (q_ref[0] + q_ref[1] + zs_ref[...])
    o_full = jnp.concatenate(
        [o_p[:, H * r:H * (r + 1)] for r in range(8)], axis=0)
    o = o_full[:N, :C] + b2_ref[...]
    m = jnp.max(o, axis=1, keepdims=True)
    lse = jnp.log(jnp.sum(jnp.exp(o - m), axis=1, keepdims=True))
    out_ref[...] = o - m - lse


_tc1 = pl.pallas_call(
    _tc1_body,
    out_shape=[
        jax.ShapeDtypeStruct((NPK, 128), jnp.float32),
        jax.ShapeDtypeStruct((NPK, 128), jnp.float32),
    ],
)

_tc2 = pl.pallas_call(
    _tc2_body,
    out_shape=jax.ShapeDtypeStruct((NPK, 128), jnp.float32),
)

_tc3 = pl.pallas_call(
    _tc3_body,
    out_shape=jax.ShapeDtypeStruct((N, C), jnp.float32),
)


@jax.jit
def kernel(x, edge_index, W1, b1, Wl, bl, W2, b2):
    def _prep(row, pad_pattern):
        r = (row * 13108) >> 24
        t = (row - r * NPK) * 8 + r
        t = jnp.concatenate(
            [t.reshape(NW, EPW),
             jnp.broadcast_to(jnp.asarray(pad_pattern), (NW, EPWP - EPW))],
            axis=1)
        return t.reshape(NW, NBLK, BLK)

    src_r = _prep(edge_index[0], PAD_SRC)
    dst_r = _prep(edge_index[1], PAD_DST)
    wl_blk = _blkdiag(Wl)
    w2_blk = _blkdiag(jnp.pad(W2, ((0, 0), (0, H - C))))
    b1_p = jnp.tile(b1, 8).reshape(1, 128)
    bl_p = jnp.tile(bl, 8).reshape(1, 128)

    deg16 = _deg_kernel(dst_r)
    deg_p = deg16.reshape(NC, NPK, 128)
    ys_p, dis_p = _tc1(x, W1, deg_p)
    ys = ys_p.reshape(NP, H)
    p = _prop_kernel(ys, src_r, dst_r)
    zs_p = _tc2(p.reshape(NC, NPK, 128), ys_p, dis_p, b1_p, wl_blk, bl_p, w2_blk)
    q = _prop_kernel(zs_p.reshape(NP, H), src_r, dst_r)
    return _tc3(q.reshape(NC, NPK, 128), zs_p, dis_p, b2.reshape(1, C))


# speedup vs baseline: 1.7060x; 1.4639x over previous
import functools

import jax
import jax.numpy as jnp
import numpy as np
from jax import lax
from jax.experimental import pallas as pl
from jax.experimental.pallas import tpu as pltpu
from jax.experimental.pallas import tpu_sc as plsc

N = 10000
E = 320000
D = 128
H = 16
C = 7
K_HOPS = 10
ALPHA = 0.1

NC = 2
NS = 16
NW = NC * NS
EPW = E // NW
BLK = 100
NBLK = EPW // BLK
NP = 10240
RPT = NP // NS
NPK = NP // 8


def _sc_mesh():
    return plsc.VectorSubcoreMesh(
        core_axis_name="c", subcore_axis_name="s", num_cores=NC, num_subcores=NS
    )


def _zero_acc_slice(zbuf, acc, s):
    def body(i, _):
        zbuf[i, :] = jnp.zeros((H,), jnp.float32)
        return 0
    lax.fori_loop(0, RPT, body, 0)
    pltpu.sync_copy(zbuf, acc.at[pl.ds(s * RPT, RPT)])


def _deg_body(dst_hbm, out_hbm, dst_v, ones_v, zbuf, acc, ssem):
    c = lax.axis_index("c")
    s = lax.axis_index("s")
    wid = c * NS + s
    _zero_acc_slice(zbuf, acc, s)

    def fill_ones(j, _):
        ones_v[j, :] = jnp.ones((H,), jnp.float32)
        return 0
    lax.fori_loop(0, BLK, fill_ones, 0)
    pltpu.sync_copy(dst_hbm.at[wid], dst_v)
    plsc.subcore_barrier()

    def blk(j, _):
        pltpu.async_copy(ones_v, acc.at[dst_v.at[j]], ssem, add=True)
        return 0
    lax.fori_loop(0, NBLK, blk, 0)

    def drain(j, _):
        pltpu.make_async_copy(ones_v, acc.at[dst_v.at[j]], ssem).wait()
        return 0
    lax.fori_loop(0, NBLK, drain, 0)
    plsc.subcore_barrier()
    pltpu.sync_copy(acc.at[pl.ds(s * RPT, RPT)], out_hbm.at[c, pl.ds(s * RPT, RPT)])


_deg_kernel = pl.kernel(
    _deg_body,
    out_type=jax.ShapeDtypeStruct((NC, NP, H), jnp.float32),
    mesh=_sc_mesh(),
    compiler_params=pltpu.CompilerParams(use_tc_tiling_on_sc=False),
    scratch_types=[
        pltpu.VMEM((NBLK, BLK), jnp.int32),
        pltpu.VMEM((BLK, H), jnp.float32),
        pltpu.VMEM((RPT, H), jnp.float32),
        pltpu.VMEM_SHARED((NP, H), jnp.float32),
        pltpu.SemaphoreType.DMA,
    ],
)


G = 5
NG = NBLK // G


def _prop_body(y_hbm, src_hbm, dst_hbm, out_hbm, src_v, dst_v, rows_v, zbuf, acc,
               gsem_a, gsem_b, ssem_a, ssem_b):
    c = lax.axis_index("c")
    s = lax.axis_index("s")
    wid = c * NS + s
    _zero_acc_slice(zbuf, acc, s)
    pltpu.sync_copy(src_hbm.at[wid], src_v)
    pltpu.sync_copy(dst_hbm.at[wid], dst_v)
    plsc.subcore_barrier()

    def gather_start(j, p, b, sem):
        pltpu.async_copy(y_hbm.at[src_v.at[j]], rows_v.at[p, b], sem)

    def gather_wait(j, p, b, sem):
        pltpu.make_async_copy(y_hbm.at[src_v.at[j]], rows_v.at[p, b], sem).wait()

    def scat_start(j, p, b, sem):
        pltpu.async_copy(rows_v.at[p, b], acc.at[dst_v.at[j]], sem, add=True)

    def scat_wait(j, p, b, sem):
        pltpu.make_async_copy(rows_v.at[p, b], acc.at[dst_v.at[j]], sem).wait()

    for b in range(G):
        gather_start(b, 0, b, gsem_a)

    def pair(g, _):
        ja = 2 * g * G
        jb = ja + G
        jn = jb + G
        for b in range(G):
            gather_wait(ja + b, 0, b, gsem_a)
        for b in range(G):
            scat_start(ja + b, 0, b, ssem_a)

        @pl.when(g > 0)
        def _():
            for b in range(G):
                scat_wait(ja - G + b, 1, b, ssem_b)
        for b in range(G):
            gather_start(jb + b, 1, b, gsem_b)
        for b in range(G):
            gather_wait(jb + b, 1, b, gsem_b)
        for b in range(G):
            scat_start(jb + b, 1, b, ssem_b)
        for b in range(G):
            scat_wait(ja + b, 0, b, ssem_a)

        @pl.when(jn < NBLK)
        def _():
            for b in range(G):
                gather_start(jn + b, 0, b, gsem_a)
        return 0

    lax.fori_loop(0, NG // 2, pair, 0)
    for b in range(G):
        scat_wait(NBLK - G + b, 1, b, ssem_b)
    plsc.subcore_barrier()
    pltpu.sync_copy(acc.at[pl.ds(s * RPT, RPT)], out_hbm.at[c, pl.ds(s * RPT, RPT)])


_prop_kernel = pl.kernel(
    _prop_body,
    out_type=jax.ShapeDtypeStruct((NC, NP, H), jnp.float32),
    mesh=_sc_mesh(),
    compiler_params=pltpu.CompilerParams(use_tc_tiling_on_sc=False),
    scratch_types=[
        pltpu.VMEM((NBLK, BLK), jnp.int32),
        pltpu.VMEM((NBLK, BLK), jnp.int32),
        pltpu.VMEM((2, G, BLK, H), jnp.float32),
        pltpu.VMEM((RPT, H), jnp.float32),
        pltpu.VMEM_SHARED((NP, H), jnp.float32),
        pltpu.SemaphoreType.DMA,
        pltpu.SemaphoreType.DMA,
        pltpu.SemaphoreType.DMA,
        pltpu.SemaphoreType.DMA,
    ],
)


def _blkdiag(w):
    return (jnp.eye(8, dtype=w.dtype)[:, None, :, None]
            * w[None, :, None, :]).reshape(H * 8, H * 8)


def _tc1_body(x_ref, w1_ref, d_ref, ys_ref, dis_ref):
    xw = jnp.dot(x_ref[...], w1_ref[...], preferred_element_type=jnp.float32)
    xw_full = jnp.concatenate([xw, jnp.zeros((NP - N, H), jnp.float32)])
    xw_p = jnp.concatenate(
        [xw_full[NPK * r:NPK * (r + 1)] for r in range(8)], axis=1)
    deg = 1.0 + d_ref[0] + d_ref[1]
    dis = lax.rsqrt(deg)
    ys_ref[...] = xw_p * dis
    dis_ref[...] = dis


def _tc2_body(p_ref, ys_ref, dis_ref, b1_ref, wl_ref, bl_ref, w2_ref, zs_ref):
    dis = dis_ref[...]
    h = dis * (p_ref[0] + p_ref[1] + ys_ref[...]) + b1_ref[...]
    h = jnp.maximum(h, 0.0)
    wl = wl_ref[...]
    bl = bl_ref[...]
    for _ in range(K_HOPS):
        t = jnp.dot(h, wl, preferred_element_type=jnp.float32) + bl
        h = ALPHA * jnp.maximum(t, 0.0) + (1.0 - ALPHA) * h
    z = jnp.dot(h, w2_ref[...], preferred_element_type=jnp.float32)
    zs_ref[...] = z * dis


def _tc3_body(q_ref, zs_ref, dis_ref, b2_ref, out_ref):
    o_p = dis_ref[...] * (q_ref[0] + q_ref[1] + zs_ref[...])
    o_full = jnp.concatenate(
        [o_p[:, H * r:H * (r + 1)] for r in range(8)], axis=0)
    o = o_full[:N, :C] + b2_ref[...]
    m = jnp.max(o, axis=1, keepdims=True)
    lse = jnp.log(jnp.sum(jnp.exp(o - m), axis=1, keepdims=True))
    out_ref[...] = o - m - lse


_tc1 = pl.pallas_call(
    _tc1_body,
    out_shape=[
        jax.ShapeDtypeStruct((NPK, 128), jnp.float32),
        jax.ShapeDtypeStruct((NPK, 128), jnp.float32),
    ],
)

_tc2 = pl.pallas_call(
    _tc2_body,
    out_shape=jax.ShapeDtypeStruct((NPK, 128), jnp.float32),
)

_tc3 = pl.pallas_call(
    _tc3_body,
    out_shape=jax.ShapeDtypeStruct((N, C), jnp.float32),
)


@jax.jit
def kernel(x, edge_index, W1, b1, Wl, bl, W2, b2):
    def _prep(row):
        r = (row * 13108) >> 24
        t = (row - r * NPK) * 8 + r
        return t.reshape(NW, NBLK, BLK)

    src_r = _prep(edge_index[0])
    dst_r = _prep(edge_index[1])
    wl_blk = _blkdiag(Wl)
    w2_blk = _blkdiag(jnp.pad(W2, ((0, 0), (0, H - C))))
    b1_p = jnp.tile(b1, 8).reshape(1, 128)
    bl_p = jnp.tile(bl, 8).reshape(1, 128)

    deg16 = _deg_kernel(dst_r)
    deg_p = deg16.reshape(NC, NPK, 128)
    ys_p, dis_p = _tc1(x, W1, deg_p)
    ys = ys_p.reshape(NP, H)
    p = _prop_kernel(ys, src_r, dst_r)
    zs_p = _tc2(p.reshape(NC, NPK, 128), ys_p, dis_p, b1_p, wl_blk, bl_p, w2_blk)
    q = _prop_kernel(zs_p.reshape(NP, H), src_r, dst_r)
    return _tc3(q.reshape(NC, NPK, 128), zs_p, dis_p, b2.reshape(1, C))


# speedup vs baseline: 1.8675x; 1.0946x over previous
import functools

import jax
import jax.numpy as jnp
import numpy as np
from jax import lax
from jax.experimental import pallas as pl
from jax.experimental.pallas import tpu as pltpu
from jax.experimental.pallas import tpu_sc as plsc

N = 10000
E = 320000
D = 128
H = 16
C = 7
K_HOPS = 10
ALPHA = 0.1

NC = 2
NS = 16
NW = NC * NS
EPW = E // NW
BLK = 100
NBLK = EPW // BLK
NP = 10240
RPT = NP // NS
NPK = NP // 8


def _sc_mesh():
    return plsc.VectorSubcoreMesh(
        core_axis_name="c", subcore_axis_name="s", num_cores=NC, num_subcores=NS
    )


def _zero_acc_slice(zbuf, acc, s):
    def body(i, _):
        zbuf[i, :] = jnp.zeros((H,), jnp.float32)
        return 0
    lax.fori_loop(0, RPT, body, 0)
    pltpu.sync_copy(zbuf, acc.at[pl.ds(s * RPT, RPT)])


def _deg_body(dst_hbm, out_hbm, dst_v, ones_v, zbuf, acc, ssem):
    c = lax.axis_index("c")
    s = lax.axis_index("s")
    wid = c * NS + s
    _zero_acc_slice(zbuf, acc, s)

    def fill_ones(j, _):
        ones_v[j, :] = jnp.ones((H,), jnp.float32)
        return 0
    lax.fori_loop(0, BLK, fill_ones, 0)
    pltpu.sync_copy(dst_hbm.at[wid], dst_v)
    plsc.subcore_barrier()

    def blk(j, _):
        pltpu.async_copy(ones_v, acc.at[dst_v.at[j]], ssem, add=True)
        return 0
    lax.fori_loop(0, NBLK, blk, 0)

    def drain(j, _):
        pltpu.make_async_copy(ones_v, acc.at[dst_v.at[j]], ssem).wait()
        return 0
    lax.fori_loop(0, NBLK, drain, 0)
    plsc.subcore_barrier()
    pltpu.sync_copy(acc.at[pl.ds(s * RPT, RPT)], out_hbm.at[c, pl.ds(s * RPT, RPT)])


_deg_kernel = pl.kernel(
    _deg_body,
    out_type=jax.ShapeDtypeStruct((NC, NP, H), jnp.float32),
    mesh=_sc_mesh(),
    compiler_params=pltpu.CompilerParams(use_tc_tiling_on_sc=False),
    scratch_types=[
        pltpu.VMEM((NBLK, BLK), jnp.int32),
        pltpu.VMEM((BLK, H), jnp.float32),
        pltpu.VMEM((RPT, H), jnp.float32),
        pltpu.VMEM_SHARED((NP, H), jnp.float32),
        pltpu.SemaphoreType.DMA,
    ],
)


G = 10
NG = NBLK // G


def _prop_body(y_hbm, src_hbm, dst_hbm, out_hbm, src_v, dst_v, rows_v, zbuf, acc,
               gsem_a, gsem_b, ssem_a, ssem_b):
    c = lax.axis_index("c")
    s = lax.axis_index("s")
    wid = c * NS + s
    _zero_acc_slice(zbuf, acc, s)
    pltpu.sync_copy(src_hbm.at[wid], src_v)
    pltpu.sync_copy(dst_hbm.at[wid], dst_v)
    plsc.subcore_barrier()

    def gather_start(j, p, b, sem):
        pltpu.async_copy(y_hbm.at[src_v.at[j]], rows_v.at[p, b], sem)

    def gather_wait(j, p, b, sem):
        pltpu.make_async_copy(y_hbm.at[src_v.at[j]], rows_v.at[p, b], sem).wait()

    def scat_start(j, p, b, sem):
        pltpu.async_copy(rows_v.at[p, b], acc.at[dst_v.at[j]], sem, add=True)

    def scat_wait(j, p, b, sem):
        pltpu.make_async_copy(rows_v.at[p, b], acc.at[dst_v.at[j]], sem).wait()

    for b in range(G):
        gather_start(b, 0, b, gsem_a)

    def pair(g, _):
        ja = 2 * g * G
        jb = ja + G
        jn = jb + G
        for b in range(G):
            gather_wait(ja + b, 0, b, gsem_a)
        for b in range(G):
            scat_start(ja + b, 0, b, ssem_a)

        @pl.when(g > 0)
        def _():
            for b in range(G):
                scat_wait(ja - G + b, 1, b, ssem_b)
        for b in range(G):
            gather_start(jb + b, 1, b, gsem_b)
        for b in range(G):
            gather_wait(jb + b, 1, b, gsem_b)
        for b in range(G):
            scat_start(jb + b, 1, b, ssem_b)
        for b in range(G):
            scat_wait(ja + b, 0, b, ssem_a)

        @pl.when(jn < NBLK)
        def _():
            for b in range(G):
                gather_start(jn + b, 0, b, gsem_a)
        return 0

    lax.fori_loop(0, NG // 2, pair, 0)
    for b in range(G):
        scat_wait(NBLK - G + b, 1, b, ssem_b)
    plsc.subcore_barrier()
    pltpu.sync_copy(acc.at[pl.ds(s * RPT, RPT)], out_hbm.at[c, pl.ds(s * RPT, RPT)])


_prop_kernel = pl.kernel(
    _prop_body,
    out_type=jax.ShapeDtypeStruct((NC, NP, H), jnp.float32),
    mesh=_sc_mesh(),
    compiler_params=pltpu.CompilerParams(use_tc_tiling_on_sc=False),
    scratch_types=[
        pltpu.VMEM((NBLK, BLK), jnp.int32),
        pltpu.VMEM((NBLK, BLK), jnp.int32),
        pltpu.VMEM((2, G, BLK, H), jnp.float32),
        pltpu.VMEM((RPT, H), jnp.float32),
        pltpu.VMEM_SHARED((NP, H), jnp.float32),
        pltpu.SemaphoreType.DMA,
        pltpu.SemaphoreType.DMA,
        pltpu.SemaphoreType.DMA,
        pltpu.SemaphoreType.DMA,
    ],
)


def _blkdiag(w):
    return (jnp.eye(8, dtype=w.dtype)[:, None, :, None]
            * w[None, :, None, :]).reshape(H * 8, H * 8)


def _tc1_body(x_ref, w1_ref, d_ref, ys_ref, dis_ref):
    xw = jnp.dot(x_ref[...], w1_ref[...], preferred_element_type=jnp.float32)
    xw_full = jnp.concatenate([xw, jnp.zeros((NP - N, H), jnp.float32)])
    xw_p = jnp.concatenate(
        [xw_full[NPK * r:NPK * (r + 1)] for r in range(8)], axis=1)
    deg = 1.0 + d_ref[0] + d_ref[1]
    dis = lax.rsqrt(deg)
    ys_ref[...] = xw_p * dis
    dis_ref[...] = dis


def _tc2_body(p_ref, ys_ref, dis_ref, b1_ref, wl_ref, bl_ref, w2_ref, zs_ref):
    dis = dis_ref[...]
    h = dis * (p_ref[0] + p_ref[1] + ys_ref[...]) + b1_ref[...]
    h = jnp.maximum(h, 0.0)
    wl = wl_ref[...]
    bl = bl_ref[...]
    for _ in range(K_HOPS):
        t = jnp.dot(h, wl, preferred_element_type=jnp.float32) + bl
        h = ALPHA * jnp.maximum(t, 0.0) + (1.0 - ALPHA) * h
    z = jnp.dot(h, w2_ref[...], preferred_element_type=jnp.float32)
    zs_ref[...] = z * dis


def _tc3_body(q_ref, zs_ref, dis_ref, b2_ref, out_ref):
    o_p = dis_ref[...] * (q_ref[0] + q_ref[1] + zs_ref[...])
    o_full = jnp.concatenate(
        [o_p[:, H * r:H * (r + 1)] for r in range(8)], axis=0)
    o = o_full[:N, :C] + b2_ref[...]
    m = jnp.max(o, axis=1, keepdims=True)
    lse = jnp.log(jnp.sum(jnp.exp(o - m), axis=1, keepdims=True))
    out_ref[...] = o - m - lse


_tc1 = pl.pallas_call(
    _tc1_body,
    out_shape=[
        jax.ShapeDtypeStruct((NPK, 128), jnp.float32),
        jax.ShapeDtypeStruct((NPK, 128), jnp.float32),
    ],
)

_tc2 = pl.pallas_call(
    _tc2_body,
    out_shape=jax.ShapeDtypeStruct((NPK, 128), jnp.float32),
)

_tc3 = pl.pallas_call(
    _tc3_body,
    out_shape=jax.ShapeDtypeStruct((N, C), jnp.float32),
)


@jax.jit
def kernel(x, edge_index, W1, b1, Wl, bl, W2, b2):
    def _prep(row):
        r = (row * 13108) >> 24
        t = (row - r * NPK) * 8 + r
        return t.reshape(NW, NBLK, BLK)

    src_r = _prep(edge_index[0])
    dst_r = _prep(edge_index[1])
    wl_blk = _blkdiag(Wl)
    w2_blk = _blkdiag(jnp.pad(W2, ((0, 0), (0, H - C))))
    b1_p = jnp.tile(b1, 8).reshape(1, 128)
    bl_p = jnp.tile(bl, 8).reshape(1, 128)

    deg16 = _deg_kernel(dst_r)
    deg_p = deg16.reshape(NC, NPK, 128)
    ys_p, dis_p = _tc1(x, W1, deg_p)
    ys = ys_p.reshape(NP, H)
    p = _prop_kernel(ys, src_r, dst_r)
    zs_p = _tc2(p.reshape(NC, NPK, 128), ys_p, dis_p, b1_p, wl_blk, bl_p, w2_blk)
    q = _prop_kernel(zs_p.reshape(NP, H), src_r, dst_r)
    return _tc3(q.reshape(NC, NPK, 128), zs_p, dis_p, b2.reshape(1, C))


# speedup vs baseline: 1.9041x; 1.0196x over previous
import functools

import jax
import jax.numpy as jnp
import numpy as np
from jax import lax
from jax.experimental import pallas as pl
from jax.experimental.pallas import tpu as pltpu
from jax.experimental.pallas import tpu_sc as plsc

N = 10000
E = 320000
D = 128
H = 16
C = 7
K_HOPS = 10
ALPHA = 0.1

NC = 2
NS = 16
NW = NC * NS
EPW = E // NW
BLK = 100
NBLK = EPW // BLK
NP = 10240
RPT = NP // NS
NPK = NP // 8


def _sc_mesh():
    return plsc.VectorSubcoreMesh(
        core_axis_name="c", subcore_axis_name="s", num_cores=NC, num_subcores=NS
    )


def _zero_acc_slice(zbuf, acc, s):
    def body(i, _):
        zbuf[i, :] = jnp.zeros((H,), jnp.float32)
        return 0
    lax.fori_loop(0, RPT, body, 0)
    pltpu.sync_copy(zbuf, acc.at[pl.ds(s * RPT, RPT)])


def _deg_body(dst_hbm, out_hbm, dst_v, ones_v, zbuf, acc, ssem):
    c = lax.axis_index("c")
    s = lax.axis_index("s")
    wid = c * NS + s
    _zero_acc_slice(zbuf, acc, s)

    def fill_ones(j, _):
        ones_v[j, :] = jnp.ones((H,), jnp.float32)
        return 0
    lax.fori_loop(0, BLK, fill_ones, 0)
    pltpu.sync_copy(dst_hbm.at[wid], dst_v)
    plsc.subcore_barrier()

    def blk(j, _):
        pltpu.async_copy(ones_v, acc.at[dst_v.at[j]], ssem, add=True)
        return 0
    lax.fori_loop(0, NBLK, blk, 0)

    def drain(j, _):
        pltpu.make_async_copy(ones_v, acc.at[dst_v.at[j]], ssem).wait()
        return 0
    lax.fori_loop(0, NBLK, drain, 0)
    plsc.subcore_barrier()
    pltpu.sync_copy(acc.at[pl.ds(s * RPT, RPT)], out_hbm.at[c, pl.ds(s * RPT, RPT)])


_deg_kernel = pl.kernel(
    _deg_body,
    out_type=jax.ShapeDtypeStruct((NC, NP, H), jnp.float32),
    mesh=_sc_mesh(),
    compiler_params=pltpu.CompilerParams(use_tc_tiling_on_sc=False),
    scratch_types=[
        pltpu.VMEM((NBLK, BLK), jnp.int32),
        pltpu.VMEM((BLK, H), jnp.float32),
        pltpu.VMEM((RPT, H), jnp.float32),
        pltpu.VMEM_SHARED((NP, H), jnp.float32),
        pltpu.SemaphoreType.DMA,
    ],
)


G = 25
NG = NBLK // G


def _prop_body(y_hbm, src_hbm, dst_hbm, out_hbm, src_v, dst_v, rows_v, zbuf, acc,
               gsem_a, gsem_b, ssem_a, ssem_b):
    c = lax.axis_index("c")
    s = lax.axis_index("s")
    wid = c * NS + s
    _zero_acc_slice(zbuf, acc, s)
    pltpu.sync_copy(src_hbm.at[wid], src_v)
    pltpu.sync_copy(dst_hbm.at[wid], dst_v)
    plsc.subcore_barrier()

    def gather_start(j, p, b, sem):
        pltpu.async_copy(y_hbm.at[src_v.at[j]], rows_v.at[p, b], sem)

    def gather_wait(j, p, b, sem):
        pltpu.make_async_copy(y_hbm.at[src_v.at[j]], rows_v.at[p, b], sem).wait()

    def scat_start(j, p, b, sem):
        pltpu.async_copy(rows_v.at[p, b], acc.at[dst_v.at[j]], sem, add=True)

    def scat_wait(j, p, b, sem):
        pltpu.make_async_copy(rows_v.at[p, b], acc.at[dst_v.at[j]], sem).wait()

    for b in range(G):
        gather_start(b, 0, b, gsem_a)

    def pair(g, _):
        ja = 2 * g * G
        jb = ja + G
        jn = jb + G
        for b in range(G):
            gather_wait(ja + b, 0, b, gsem_a)
        for b in range(G):
            scat_start(ja + b, 0, b, ssem_a)

        @pl.when(g > 0)
        def _():
            for b in range(G):
                scat_wait(ja - G + b, 1, b, ssem_b)
        for b in range(G):
            gather_start(jb + b, 1, b, gsem_b)
        for b in range(G):
            gather_wait(jb + b, 1, b, gsem_b)
        for b in range(G):
            scat_start(jb + b, 1, b, ssem_b)
        for b in range(G):
            scat_wait(ja + b, 0, b, ssem_a)

        @pl.when(jn < NBLK)
        def _():
            for b in range(G):
                gather_start(jn + b, 0, b, gsem_a)
        return 0

    lax.fori_loop(0, NG // 2, pair, 0)
    for b in range(G):
        scat_wait(NBLK - G + b, 1, b, ssem_b)
    plsc.subcore_barrier()
    pltpu.sync_copy(acc.at[pl.ds(s * RPT, RPT)], out_hbm.at[c, pl.ds(s * RPT, RPT)])


_prop_kernel = pl.kernel(
    _prop_body,
    out_type=jax.ShapeDtypeStruct((NC, NP, H), jnp.float32),
    mesh=_sc_mesh(),
    compiler_params=pltpu.CompilerParams(use_tc_tiling_on_sc=False),
    scratch_types=[
        pltpu.VMEM((NBLK, BLK), jnp.int32),
        pltpu.VMEM((NBLK, BLK), jnp.int32),
        pltpu.VMEM((2, G, BLK, H), jnp.float32),
        pltpu.VMEM((RPT, H), jnp.float32),
        pltpu.VMEM_SHARED((NP, H), jnp.float32),
        pltpu.SemaphoreType.DMA,
        pltpu.SemaphoreType.DMA,
        pltpu.SemaphoreType.DMA,
        pltpu.SemaphoreType.DMA,
    ],
)


def _blkdiag(w):
    return (jnp.eye(8, dtype=w.dtype)[:, None, :, None]
            * w[None, :, None, :]).reshape(H * 8, H * 8)


def _tc1_body(x_ref, w1_ref, d_ref, ys_ref, dis_ref):
    xw = jnp.dot(x_ref[...], w1_ref[...], preferred_element_type=jnp.float32)
    xw_full = jnp.concatenate([xw, jnp.zeros((NP - N, H), jnp.float32)])
    xw_p = jnp.concatenate(
        [xw_full[NPK * r:NPK * (r + 1)] for r in range(8)], axis=1)
    deg = 1.0 + d_ref[0] + d_ref[1]
    dis = lax.rsqrt(deg)
    ys_ref[...] = xw_p * dis
    dis_ref[...] = dis


def _tc2_body(p_ref, ys_ref, dis_ref, b1_ref, wl_ref, bl_ref, w2_ref, zs_ref):
    dis = dis_ref[...]
    h = dis * (p_ref[0] + p_ref[1] + ys_ref[...]) + b1_ref[...]
    h = jnp.maximum(h, 0.0)
    wl = wl_ref[...]
    bl = bl_ref[...]
    for _ in range(K_HOPS):
        t = jnp.dot(h, wl, preferred_element_type=jnp.float32) + bl
        h = ALPHA * jnp.maximum(t, 0.0) + (1.0 - ALPHA) * h
    z = jnp.dot(h, w2_ref[...], preferred_element_type=jnp.float32)
    zs_ref[...] = z * dis


def _tc3_body(q_ref, zs_ref, dis_ref, b2_ref, out_ref):
    o_p = dis_ref[...] * (q_ref[0] + q_ref[1] + zs_ref[...])
    o_full = jnp.concatenate(
        [o_p[:, H * r:H * (r + 1)] for r in range(8)], axis=0)
    o = o_full[:N, :C] + b2_ref[...]
    m = jnp.max(o, axis=1, keepdims=True)
    lse = jnp.log(jnp.sum(jnp.exp(o - m), axis=1, keepdims=True))
    out_ref[...] = o - m - lse


_tc1 = pl.pallas_call(
    _tc1_body,
    out_shape=[
        jax.ShapeDtypeStruct((NPK, 128), jnp.float32),
        jax.ShapeDtypeStruct((NPK, 128), jnp.float32),
    ],
)

_tc2 = pl.pallas_call(
    _tc2_body,
    out_shape=jax.ShapeDtypeStruct((NPK, 128), jnp.float32),
)

_tc3 = pl.pallas_call(
    _tc3_body,
    out_shape=jax.ShapeDtypeStruct((N, C), jnp.float32),
)


@jax.jit
def kernel(x, edge_index, W1, b1, Wl, bl, W2, b2):
    def _prep(row):
        r = (row * 13108) >> 24
        t = (row - r * NPK) * 8 + r
        return t.reshape(NW, NBLK, BLK)

    src_r = _prep(edge_index[0])
    dst_r = _prep(edge_index[1])
    wl_blk = _blkdiag(Wl)
    w2_blk = _blkdiag(jnp.pad(W2, ((0, 0), (0, H - C))))
    b1_p = jnp.tile(b1, 8).reshape(1, 128)
    bl_p = jnp.tile(bl, 8).reshape(1, 128)

    deg16 = _deg_kernel(dst_r)
    deg_p = deg16.reshape(NC, NPK, 128)
    ys_p, dis_p = _tc1(x, W1, deg_p)
    ys = ys_p.reshape(NP, H)
    p = _prop_kernel(ys, src_r, dst_r)
    zs_p = _tc2(p.reshape(NC, NPK, 128), ys_p, dis_p, b1_p, wl_blk, bl_p, w2_blk)
    q = _prop_kernel(zs_p.reshape(NP, H), src_r, dst_r)
    return _tc3(q.reshape(NC, NPK, 128), zs_p, dis_p, b2.reshape(1, C))


# speedup vs baseline: 1.9218x; 1.0093x over previous
import functools

import jax
import jax.numpy as jnp
import numpy as np
from jax import lax
from jax.experimental import pallas as pl
from jax.experimental.pallas import tpu as pltpu
from jax.experimental.pallas import tpu_sc as plsc

N = 10000
E = 320000
D = 128
H = 16
C = 7
K_HOPS = 10
ALPHA = 0.1

NC = 2
NS = 16
NW = NC * NS
EPW = E // NW
BLK = 125
NBLK = EPW // BLK
NP = 10240
RPT = NP // NS
NPK = NP // 8


def _sc_mesh():
    return plsc.VectorSubcoreMesh(
        core_axis_name="c", subcore_axis_name="s", num_cores=NC, num_subcores=NS
    )


def _zero_acc_slice(zbuf, acc, s):
    def body(i, _):
        zbuf[i, :] = jnp.zeros((H,), jnp.float32)
        return 0
    lax.fori_loop(0, RPT, body, 0)
    pltpu.sync_copy(zbuf, acc.at[pl.ds(s * RPT, RPT)])


def _deg_body(dst_hbm, out_hbm, dst_v, ones_v, zbuf, acc, ssem):
    c = lax.axis_index("c")
    s = lax.axis_index("s")
    wid = c * NS + s
    _zero_acc_slice(zbuf, acc, s)

    def fill_ones(j, _):
        ones_v[j, :] = jnp.ones((H,), jnp.float32)
        return 0
    lax.fori_loop(0, BLK, fill_ones, 0)
    pltpu.sync_copy(dst_hbm.at[wid], dst_v)
    plsc.subcore_barrier()

    def blk(j, _):
        pltpu.async_copy(ones_v, acc.at[dst_v.at[j]], ssem, add=True)
        return 0
    lax.fori_loop(0, NBLK, blk, 0)

    def drain(j, _):
        pltpu.make_async_copy(ones_v, acc.at[dst_v.at[j]], ssem).wait()
        return 0
    lax.fori_loop(0, NBLK, drain, 0)
    plsc.subcore_barrier()
    pltpu.sync_copy(acc.at[pl.ds(s * RPT, RPT)], out_hbm.at[c, pl.ds(s * RPT, RPT)])


_deg_kernel = pl.kernel(
    _deg_body,
    out_type=jax.ShapeDtypeStruct((NC, NP, H), jnp.float32),
    mesh=_sc_mesh(),
    compiler_params=pltpu.CompilerParams(use_tc_tiling_on_sc=False),
    scratch_types=[
        pltpu.VMEM((NBLK, BLK), jnp.int32),
        pltpu.VMEM((BLK, H), jnp.float32),
        pltpu.VMEM((RPT, H), jnp.float32),
        pltpu.VMEM_SHARED((NP, H), jnp.float32),
        pltpu.SemaphoreType.DMA,
    ],
)


G = 20
NG = NBLK // G


def _prop_body(y_hbm, src_hbm, dst_hbm, out_hbm, src_v, dst_v, rows_v, zbuf, acc,
               gsem_a, gsem_b, ssem_a, ssem_b):
    c = lax.axis_index("c")
    s = lax.axis_index("s")
    wid = c * NS + s
    _zero_acc_slice(zbuf, acc, s)
    pltpu.sync_copy(src_hbm.at[wid], src_v)
    pltpu.sync_copy(dst_hbm.at[wid], dst_v)
    plsc.subcore_barrier()

    def gather_start(j, p, b, sem):
        pltpu.async_copy(y_hbm.at[src_v.at[j]], rows_v.at[p, b], sem)

    def gather_wait(j, p, b, sem):
        pltpu.make_async_copy(y_hbm.at[src_v.at[j]], rows_v.at[p, b], sem).wait()

    def scat_start(j, p, b, sem):
        pltpu.async_copy(rows_v.at[p, b], acc.at[dst_v.at[j]], sem, add=True)

    def scat_wait(j, p, b, sem):
        pltpu.make_async_copy(rows_v.at[p, b], acc.at[dst_v.at[j]], sem).wait()

    for b in range(G):
        gather_start(b, 0, b, gsem_a)

    def pair(g, _):
        ja = 2 * g * G
        jb = ja + G
        jn = jb + G
        for b in range(G):
            gather_wait(ja + b, 0, b, gsem_a)
        for b in range(G):
            scat_start(ja + b, 0, b, ssem_a)

        @pl.when(g > 0)
        def _():
            for b in range(G):
                scat_wait(ja - G + b, 1, b, ssem_b)
        for b in range(G):
            gather_start(jb + b, 1, b, gsem_b)
        for b in range(G):
            gather_wait(jb + b, 1, b, gsem_b)
        for b in range(G):
            scat_start(jb + b, 1, b, ssem_b)
        for b in range(G):
            scat_wait(ja + b, 0, b, ssem_a)

        @pl.when(jn < NBLK)
        def _():
            for b in range(G):
                gather_start(jn + b, 0, b, gsem_a)
        return 0

    lax.fori_loop(0, NG // 2, pair, 0)
    for b in range(G):
        scat_wait(NBLK - G + b, 1, b, ssem_b)
    plsc.subcore_barrier()
    pltpu.sync_copy(acc.at[pl.ds(s * RPT, RPT)], out_hbm.at[c, pl.ds(s * RPT, RPT)])


_prop_kernel = pl.kernel(
    _prop_body,
    out_type=jax.ShapeDtypeStruct((NC, NP, H), jnp.float32),
    mesh=_sc_mesh(),
    compiler_params=pltpu.CompilerParams(use_tc_tiling_on_sc=False),
    scratch_types=[
        pltpu.VMEM((NBLK, BLK), jnp.int32),
        pltpu.VMEM((NBLK, BLK), jnp.int32),
        pltpu.VMEM((2, G, BLK, H), jnp.float32),
        pltpu.VMEM((RPT, H), jnp.float32),
        pltpu.VMEM_SHARED((NP, H), jnp.float32),
        pltpu.SemaphoreType.DMA,
        pltpu.SemaphoreType.DMA,
        pltpu.SemaphoreType.DMA,
        pltpu.SemaphoreType.DMA,
    ],
)


def _blkdiag(w):
    return (jnp.eye(8, dtype=w.dtype)[:, None, :, None]
            * w[None, :, None, :]).reshape(H * 8, H * 8)


def _tc1_body(x_ref, w1_ref, d_ref, ys_ref, dis_ref):
    xw = jnp.dot(x_ref[...], w1_ref[...], preferred_element_type=jnp.float32)
    xw_full = jnp.concatenate([xw, jnp.zeros((NP - N, H), jnp.float32)])
    xw_p = jnp.concatenate(
        [xw_full[NPK * r:NPK * (r + 1)] for r in range(8)], axis=1)
    deg = 1.0 + d_ref[0] + d_ref[1]
    dis = lax.rsqrt(deg)
    ys_ref[...] = xw_p * dis
    dis_ref[...] = dis


def _tc2_body(p_ref, ys_ref, dis_ref, b1_ref, wl_ref, bl_ref, w2_ref, zs_ref):
    dis = dis_ref[...]
    h = dis * (p_ref[0] + p_ref[1] + ys_ref[...]) + b1_ref[...]
    h = jnp.maximum(h, 0.0)
    wl = wl_ref[...]
    bl = bl_ref[...]
    for _ in range(K_HOPS):
        t = jnp.dot(h, wl, preferred_element_type=jnp.float32) + bl
        h = ALPHA * jnp.maximum(t, 0.0) + (1.0 - ALPHA) * h
    z = jnp.dot(h, w2_ref[...], preferred_element_type=jnp.float32)
    zs_ref[...] = z * dis


def _tc3_body(q_ref, zs_ref, dis_ref, b2_ref, out_ref):
    o_p = dis_ref[...] * (q_ref[0] + q_ref[1] + zs_ref[...])
    o_full = jnp.concatenate(
        [o_p[:, H * r:H * (r + 1)] for r in range(8)], axis=0)
    o = o_full[:N, :C] + b2_ref[...]
    m = jnp.max(o, axis=1, keepdims=True)
    lse = jnp.log(jnp.sum(jnp.exp(o - m), axis=1, keepdims=True))
    out_ref[...] = o - m - lse


_tc1 = pl.pallas_call(
    _tc1_body,
    out_shape=[
        jax.ShapeDtypeStruct((NPK, 128), jnp.float32),
        jax.ShapeDtypeStruct((NPK, 128), jnp.float32),
    ],
)

_tc2 = pl.pallas_call(
    _tc2_body,
    out_shape=jax.ShapeDtypeStruct((NPK, 128), jnp.float32),
)

_tc3 = pl.pallas_call(
    _tc3_body,
    out_shape=jax.ShapeDtypeStruct((N, C), jnp.float32),
)


@jax.jit
def kernel(x, edge_index, W1, b1, Wl, bl, W2, b2):
    def _prep(row):
        r = (row * 13108) >> 24
        t = (row - r * NPK) * 8 + r
        return t.reshape(NW, NBLK, BLK)

    src_r = _prep(edge_index[0])
    dst_r = _prep(edge_index[1])
    wl_blk = _blkdiag(Wl)
    w2_blk = _blkdiag(jnp.pad(W2, ((0, 0), (0, H - C))))
    b1_p = jnp.tile(b1, 8).reshape(1, 128)
    bl_p = jnp.tile(bl, 8).reshape(1, 128)

    deg16 = _deg_kernel(dst_r)
    deg_p = deg16.reshape(NC, NPK, 128)
    ys_p, dis_p = _tc1(x, W1, deg_p)
    ys = ys_p.reshape(NP, H)
    p = _prop_kernel(ys, src_r, dst_r)
    zs_p = _tc2(p.reshape(NC, NPK, 128), ys_p, dis_p, b1_p, wl_blk, bl_p, w2_blk)
    q = _prop_kernel(zs_p.reshape(NP, H), src_r, dst_r)
    return _tc3(q.reshape(NC, NPK, 128), zs_p, dis_p, b2.reshape(1, C))


# speedup vs baseline: 2.1605x; 1.1242x over previous
import functools

import jax
import jax.numpy as jnp
import numpy as np
from jax import lax
from jax.experimental import pallas as pl
from jax.experimental.pallas import tpu as pltpu
from jax.experimental.pallas import tpu_sc as plsc

N = 10000
E = 320000
D = 128
H = 16
C = 7
K_HOPS = 10
ALPHA = 0.1

NC = 2
NS = 16
NW = NC * NS
EPW = E // NW
BLK = 125
NBLK = EPW // BLK
NP = 10240
RPT = NP // NS
NPK = NP // 8


def _sc_mesh():
    return plsc.VectorSubcoreMesh(
        core_axis_name="c", subcore_axis_name="s", num_cores=NC, num_subcores=NS
    )


def _zero_acc_slice(zbuf, acc, s):
    def body(i, _):
        zbuf[i, :] = jnp.zeros((H,), jnp.float32)
        return 0
    lax.fori_loop(0, RPT, body, 0)
    pltpu.sync_copy(zbuf, acc.at[pl.ds(s * RPT, RPT)])


def _deg_body(dst_hbm, out_hbm, dst_v, ones_v, zbuf, acc, ssem):
    c = lax.axis_index("c")
    s = lax.axis_index("s")
    wid = c * NS + s
    _zero_acc_slice(zbuf, acc, s)

    def fill_ones(j, _):
        ones_v[j, :] = jnp.ones((H,), jnp.float32)
        return 0
    lax.fori_loop(0, BLK, fill_ones, 0)
    pltpu.sync_copy(dst_hbm.at[wid], dst_v)
    plsc.subcore_barrier()

    def blk(j, _):
        pltpu.async_copy(ones_v, acc.at[dst_v.at[j]], ssem, add=True)
        return 0
    lax.fori_loop(0, NBLK, blk, 0)

    def drain(j, _):
        pltpu.make_async_copy(ones_v, acc.at[dst_v.at[j]], ssem).wait()
        return 0
    lax.fori_loop(0, NBLK, drain, 0)
    plsc.subcore_barrier()
    pltpu.sync_copy(acc.at[pl.ds(s * RPT, RPT)], out_hbm.at[c, pl.ds(s * RPT, RPT)])


_deg_kernel = pl.kernel(
    _deg_body,
    out_type=jax.ShapeDtypeStruct((NC, NP, H), jnp.float32),
    mesh=_sc_mesh(),
    compiler_params=pltpu.CompilerParams(use_tc_tiling_on_sc=False),
    scratch_types=[
        pltpu.VMEM((NBLK, BLK), jnp.int32),
        pltpu.VMEM((BLK, H), jnp.float32),
        pltpu.VMEM((RPT, H), jnp.float32),
        pltpu.VMEM_SHARED((NP, H), jnp.float32),
        pltpu.SemaphoreType.DMA,
    ],
)


G = 20
NG = NBLK // G


def _prop_body(y_hbm, src_hbm, dst_hbm, out_hbm, src_v, dst_v, rows_v, zbuf, acc,
               gsem_a, gsem_b, ssem_a, ssem_b):
    c = lax.axis_index("c")
    s = lax.axis_index("s")
    wid = c * NS + s
    _zero_acc_slice(zbuf, acc, s)
    pltpu.sync_copy(src_hbm.at[wid], src_v)
    pltpu.sync_copy(dst_hbm.at[wid], dst_v)
    plsc.subcore_barrier()

    def gather_start(j, p, b, sem):
        pltpu.async_copy(y_hbm.at[src_v.at[j]], rows_v.at[p, b], sem)

    def gather_wait(j, p, b, sem):
        pltpu.make_async_copy(y_hbm.at[src_v.at[j]], rows_v.at[p, b], sem).wait()

    def scat_start(j, p, b, sem):
        pltpu.async_copy(rows_v.at[p, b], acc.at[dst_v.at[j]], sem, add=True)

    def scat_wait(j, p, b, sem):
        pltpu.make_async_copy(rows_v.at[p, b], acc.at[dst_v.at[j]], sem).wait()

    for b in range(G):
        gather_start(b, 0, b, gsem_a)

    def pair(g, _):
        ja = 2 * g * G
        jb = ja + G
        jn = jb + G
        for b in range(G):
            gather_wait(ja + b, 0, b, gsem_a)
        for b in range(G):
            scat_start(ja + b, 0, b, ssem_a)

        @pl.when(g > 0)
        def _():
            for b in range(G):
                scat_wait(ja - G + b, 1, b, ssem_b)
        for b in range(G):
            gather_start(jb + b, 1, b, gsem_b)
        for b in range(G):
            gather_wait(jb + b, 1, b, gsem_b)
        for b in range(G):
            scat_start(jb + b, 1, b, ssem_b)
        for b in range(G):
            scat_wait(ja + b, 0, b, ssem_a)

        @pl.when(jn < NBLK)
        def _():
            for b in range(G):
                gather_start(jn + b, 0, b, gsem_a)
        return 0

    lax.fori_loop(0, NG // 2, pair, 0)
    for b in range(G):
        scat_wait(NBLK - G + b, 1, b, ssem_b)
    plsc.subcore_barrier()
    pltpu.sync_copy(acc.at[pl.ds(s * RPT, RPT)], out_hbm.at[c, pl.ds(s * RPT, RPT)])


_prop_kernel = pl.kernel(
    _prop_body,
    out_type=jax.ShapeDtypeStruct((NC, NP, H), jnp.float32),
    mesh=_sc_mesh(),
    compiler_params=pltpu.CompilerParams(use_tc_tiling_on_sc=False),
    scratch_types=[
        pltpu.VMEM((NBLK, BLK), jnp.int32),
        pltpu.VMEM((NBLK, BLK), jnp.int32),
        pltpu.VMEM((2, G, BLK, H), jnp.float32),
        pltpu.VMEM((RPT, H), jnp.float32),
        pltpu.VMEM_SHARED((NP, H), jnp.float32),
        pltpu.SemaphoreType.DMA,
        pltpu.SemaphoreType.DMA,
        pltpu.SemaphoreType.DMA,
        pltpu.SemaphoreType.DMA,
    ],
)


def _blkdiag(w):
    return (jnp.eye(8, dtype=w.dtype)[:, None, :, None]
            * w[None, :, None, :]).reshape(H * 8, H * 8)


def _tc0_body(x_ref, w1_ref, xw_ref):
    xw = jnp.dot(x_ref[...], w1_ref[...], preferred_element_type=jnp.float32)
    xw_full = jnp.concatenate([xw, jnp.zeros((NP - N, H), jnp.float32)])
    xw_ref[...] = jnp.concatenate(
        [xw_full[NPK * r:NPK * (r + 1)] for r in range(8)], axis=1)


def _tc1_body(xw_ref, d_ref, ys_ref, dis_ref):
    deg = 1.0 + d_ref[0] + d_ref[1]
    dis = lax.rsqrt(deg)
    ys_ref[...] = xw_ref[...] * dis
    dis_ref[...] = dis


def _tc2_body(p_ref, ys_ref, dis_ref, b1_ref, wl_ref, bl_ref, w2_ref, zs_ref):
    dis = dis_ref[...]
    h = dis * (p_ref[0] + p_ref[1] + ys_ref[...]) + b1_ref[...]
    h = jnp.maximum(h, 0.0)
    wl = wl_ref[...]
    bl = bl_ref[...]
    for _ in range(K_HOPS):
        t = jnp.dot(h, wl, preferred_element_type=jnp.float32) + bl
        h = ALPHA * jnp.maximum(t, 0.0) + (1.0 - ALPHA) * h
    z = jnp.dot(h, w2_ref[...], preferred_element_type=jnp.float32)
    zs_ref[...] = z * dis


def _tc3_body(q_ref, zs_ref, dis_ref, b2_ref, out_ref):
    o_p = dis_ref[...] * (q_ref[0] + q_ref[1] + zs_ref[...])
    o_t = o_p.T
    o_full = jnp.concatenate(
        [o_t[H * r:H * r + C] for r in range(8)], axis=1)
    o = o_full[:, :N] + b2_ref[...]
    m = jnp.max(o, axis=0, keepdims=True)
    lse = jnp.log(jnp.sum(jnp.exp(o - m), axis=0, keepdims=True))
    out_ref[...] = o - m - lse


_tc0 = pl.pallas_call(
    _tc0_body,
    out_shape=jax.ShapeDtypeStruct((NPK, 128), jnp.float32),
)

_tc1 = pl.pallas_call(
    _tc1_body,
    out_shape=[
        jax.ShapeDtypeStruct((NPK, 128), jnp.float32),
        jax.ShapeDtypeStruct((NPK, 128), jnp.float32),
    ],
)

_tc2 = pl.pallas_call(
    _tc2_body,
    out_shape=jax.ShapeDtypeStruct((NPK, 128), jnp.float32),
)

_tc3 = pl.pallas_call(
    _tc3_body,
    out_shape=jax.ShapeDtypeStruct((C, N), jnp.float32),
)


@jax.jit
def kernel(x, edge_index, W1, b1, Wl, bl, W2, b2):
    def _prep(row):
        r = (row * 13108) >> 24
        t = (row - r * NPK) * 8 + r
        return t.reshape(NW, NBLK, BLK)

    src_r = _prep(edge_index[0])
    dst_r = _prep(edge_index[1])
    wl_blk = _blkdiag(Wl)
    w2_blk = _blkdiag(jnp.pad(W2, ((0, 0), (0, H - C))))
    b1_p = jnp.tile(b1, 8).reshape(1, 128)
    bl_p = jnp.tile(bl, 8).reshape(1, 128)

    deg16 = _deg_kernel(dst_r)
    xw_p = _tc0(x, W1)
    deg_p = deg16.reshape(NC, NPK, 128)
    ys_p, dis_p = _tc1(xw_p, deg_p)
    ys = ys_p.reshape(NP, H)
    p = _prop_kernel(ys, src_r, dst_r)
    zs_p = _tc2(p.reshape(NC, NPK, 128), ys_p, dis_p, b1_p, wl_blk, bl_p, w2_blk)
    q = _prop_kernel(zs_p.reshape(NP, H), src_r, dst_r)
    return _tc3(q.reshape(NC, NPK, 128), zs_p, dis_p, b2.reshape(C, 1)).T
